# baseline (speedup 1.0000x reference)
import functools

import jax
import jax.numpy as jnp
from jax import lax
from jax.experimental import pallas as pl
from jax.experimental.pallas import tpu as pltpu
from jax.experimental.pallas import tpu_sc as plsc

_B = 16384
_EMB = 64
_PHYS = 1000064
_WPW = 245
_RANGE = _WPW * 128
_CW = 256
_NCH = 128
_NBUCK = 8
_BW_SHIFT = 12
_LCAP = 1280
_BCAP = 256
_CCAP = 64
_OUT_ROWS = 2 * _B + 1024
_DUMP = 2 * _B


def _sc_gather(tableT, tgt, ctx):
    info = plsc.get_sparse_core_info()
    nc, ns = info.num_cores, info.num_subcores
    mesh = plsc.VectorSubcoreMesh(core_axis_name="c", subcore_axis_name="s")

    scratch = [
        pltpu.VMEM((4096,), jnp.int32),
        pltpu.VMEM((_LCAP,), jnp.int32),
        pltpu.VMEM((_LCAP,), jnp.int32),
        pltpu.VMEM((_CCAP,), jnp.int32),
        pltpu.VMEM((_CCAP,), jnp.int32),
    ]
    scratch += [pltpu.VMEM((_BCAP,), jnp.int32) for _ in range(2 * _NBUCK)]
    scratch += [pltpu.VMEM((_EMB, _CW), jnp.float32) for _ in range(4)]
    scratch += [pltpu.VMEM((_CCAP, 128), jnp.float32) for _ in range(2)]
    scratch += [pltpu.SemaphoreType.DMA] * 5

    @functools.partial(
        pl.kernel,
        mesh=mesh,
        compiler_params=pltpu.CompilerParams(needs_layout_passes=False),
        out_type=jax.ShapeDtypeStruct((_OUT_ROWS, 128), jnp.float32),
        scratch_types=scratch,
    )
    def k(tgt_hbm, ctx_hbm, tT_hbm, out_hbm, seg_v, mid_v, mbp_v, ccol_v,
          cbp_v, *rest):
        bid = rest[0:_NBUCK]
        bbp = rest[_NBUCK:2 * _NBUCK]
        bufs = rest[2 * _NBUCK:2 * _NBUCK + 4]
        scbs = rest[2 * _NBUCK + 4:2 * _NBUCK + 6]
        sems = rest[2 * _NBUCK + 6:2 * _NBUCK + 10]
        sem_s = rest[2 * _NBUCK + 10]

        wid = lax.axis_index("s") * nc + lax.axis_index("c")
        lo = wid * _RANGE
        hi = lo + _RANGE
        iota16 = lax.iota(jnp.int32, 16)
        dump_vec = jnp.full((16,), _DUMP, jnp.int32)

        def dstart(c):
            return pl.multiple_of(
                jnp.minimum(lo + c * _CW, _PHYS - _CW), 128)

        for r in range(4):
            pltpu.async_copy(
                tT_hbm.at[:, pl.ds(dstart(jnp.int32(r)), _CW)],
                bufs[r], sems[r])

        def scan_seg(idx_hbm, seg_base, bpos_off, cnt):
            pltpu.sync_copy(idx_hbm.at[pl.ds(seg_base, 4096)], seg_v)

            def grp(g, cnt):
                v = seg_v[pl.ds(g * 16, 16)]
                mask = (v >= lo) & (v < hi)
                mcount = jnp.sum(mask.astype(jnp.int32))
                plsc.store_compressed(mid_v.at[pl.ds(cnt, 16)], v, mask=mask)
                bpos = iota16 + (bpos_off + seg_base + g * 16)
                plsc.store_compressed(mbp_v.at[pl.ds(cnt, 16)], bpos,
                                      mask=mask)
                return cnt + mcount

            return lax.fori_loop(0, 256, grp, cnt)

        cnt = lax.fori_loop(
            0, 4, lambda s, cnt: scan_seg(tgt_hbm, s * 4096, 0, cnt),
            jnp.int32(0))
        cnt = lax.fori_loop(
            0, 4, lambda s, cnt: scan_seg(ctx_hbm, s * 4096, _B, cnt), cnt)

        n_lg = (cnt + 15) >> 4

        def distr(g, bcnts):
            vmask = (iota16 + g * 16) < cnt
            mv = mid_v[pl.ds(g * 16, 16)]
            bv = mbp_v[pl.ds(g * 16, 16)]
            bq = lax.shift_right_logical(mv - lo, _BW_SHIFT)
            out = []
            for b in range(_NBUCK):
                mb = (bq == b) & vmask
                plsc.store_compressed(bid[b].at[pl.ds(bcnts[b], 16)], mv,
                                      mask=mb)
                plsc.store_compressed(bbp[b].at[pl.ds(bcnts[b], 16)], bv,
                                      mask=mb)
                out.append(bcnts[b] + jnp.sum(mb.astype(jnp.int32)))
            return tuple(out)

        bcnts = lax.fori_loop(0, n_lg, distr, (jnp.int32(0),) * _NBUCK)

        def drain_n(n):
            def dr(_, carry):
                pltpu.make_async_copy(
                    out_hbm.at[pl.ds(0, 16)], scbs[0].at[pl.ds(0, 16)],
                    sem_s).wait()
                return carry

            lax.fori_loop(0, n, dr, jnp.int32(0))

        nsg_slot = [jnp.int32(0), jnp.int32(0)]
        for b in range(_NBUCK):
            n_bg_cap = (bcnts[b] + 15) >> 4

            def superstep(ss, carry, _b=b, _nbg=n_bg_cap):
                ns0, ns1 = carry
                nsg = [ns0, ns1]
                for slot in range(4):
                    c = _b * 16 + ss * 4 + slot
                    buf, sem = bufs[slot], sems[slot]
                    scb = scbs[slot % 2]
                    pltpu.make_async_copy(
                        tT_hbm.at[:, pl.ds(dstart(c), _CW)], buf, sem).wait()
                    ds0 = dstart(c)

                    def grp2(g, mcnt, _bb=_b):
                        vmask = (iota16 + g * 16) < bcnts[_bb]
                        mv = bid[_bb][pl.ds(g * 16, 16)]
                        bv = bbp[_bb][pl.ds(g * 16, 16)]
                        inm = (mv >= ds0) & (mv < ds0 + _CW) & vmask
                        m2 = jnp.sum(inm.astype(jnp.int32))
                        plsc.store_compressed(
                            ccol_v.at[pl.ds(mcnt, 16)], mv - ds0, mask=inm)
                        plsc.store_compressed(
                            cbp_v.at[pl.ds(mcnt, 16)], bv, mask=inm)
                        return mcnt + m2

                    mcnt = lax.fori_loop(0, _nbg, grp2, jnp.int32(0))
                    cbp_v[pl.ds(mcnt, 16)] = dump_vec
                    n_mg = (mcnt + 15) >> 4

                    drain_n(nsg[slot % 2])

                    def mgrp(mg, carry2):
                        colv = ccol_v[pl.ds(mg * 16, 16)]
                        rowv = iota16 + mg * 16
                        vm = rowv < mcnt

                        def ecol(e, c3):
                            e_vec = jnp.full((16,), e, jnp.int32)
                            vals = plsc.load_gather(buf, [e_vec, colv],
                                                    mask=vm)
                            plsc.store_scatter(scb, [rowv, e_vec], vals,
                                               mask=vm)
                            return c3

                        lax.fori_loop(0, _EMB, ecol, jnp.int32(0), unroll=4)
                        return carry2

                    lax.fori_loop(0, n_mg, mgrp, jnp.int32(0))

                    def scat(sg, carry2):
                        bvec = cbp_v[pl.ds(sg * 16, 16)]
                        pltpu.async_copy(
                            scb.at[pl.ds(sg * 16, 16)], out_hbm.at[bvec],
                            sem_s)
                        return carry2

                    lax.fori_loop(0, n_mg, scat, jnp.int32(0))
                    nsg[slot % 2] = n_mg

                    @pl.when(c + 4 < _NCH)
                    def _():
                        pltpu.async_copy(
                            tT_hbm.at[:, pl.ds(dstart(c + 4), _CW)], buf, sem)

                return (nsg[0], nsg[1])

            nsg_slot = list(
                lax.fori_loop(0, 4, superstep,
                              (nsg_slot[0], nsg_slot[1])))

        drain_n(nsg_slot[0])
        drain_n(nsg_slot[1])

    return k(tgt, ctx, tableT)


def _tc_mlp(xy, w1, b1, w2, b2, w3, b3):
    blk = 1024
    n_blk = _B // blk

    def body(x_ref, y_ref, w1_ref, b1_ref, w2_ref, b2_ref, w3_ref, b3_ref,
             o_ref):
        shared = x_ref[:, :_EMB] * y_ref[:, :_EMB]
        h1 = jnp.maximum(
            jnp.dot(shared, w1_ref[...], preferred_element_type=jnp.float32)
            + b1_ref[...], 0.0)
        h2 = jnp.maximum(
            jnp.dot(h1, w2_ref[...], preferred_element_type=jnp.float32)
            + b2_ref[...], 0.0)
        z = jnp.dot(h2, w3_ref[...], preferred_element_type=jnp.float32) \
            + b3_ref[...]
        o_ref[...] = jax.nn.sigmoid(z)

    zero2 = lambda i: (0, 0)
    return pl.pallas_call(
        body,
        grid=(n_blk,),
        in_specs=[
            pl.BlockSpec((blk, 128), lambda i: (i, 0)),
            pl.BlockSpec((blk, 128), lambda i: (i + n_blk, 0)),
            pl.BlockSpec((_EMB, 64), zero2),
            pl.BlockSpec((1, 64), zero2),
            pl.BlockSpec((64, 32), zero2),
            pl.BlockSpec((1, 32), zero2),
            pl.BlockSpec((32, 1), zero2),
            pl.BlockSpec((1, 1), zero2),
        ],
        out_specs=pl.BlockSpec((blk, 1), lambda i: (i, 0)),
        out_shape=jax.ShapeDtypeStruct((_B, 1), jnp.float32),
    )(xy, xy, w1, b1, w2, b2, w3, b3)


def kernel(target_word, context_word, table, W1, b1, W2, b2, W3, b3):
    xy = _sc_gather(table.T, target_word.astype(jnp.int32),
                    context_word.astype(jnp.int32))
    out = _tc_mlp(xy, W1, b1.reshape(1, -1), W2, b2.reshape(1, -1), W3,
                  b3.reshape(1, 1))
    return jnp.reshape(out, (-1,))


# speedup vs baseline: 2.3313x; 2.3313x over previous
import functools

import jax
import jax.numpy as jnp
from jax import lax
from jax.experimental import pallas as pl
from jax.experimental.pallas import tpu as pltpu
from jax.experimental.pallas import tpu_sc as plsc

_BATCH = 16384
_EMB = 64
_CHUNK = 128


def _sc_gather(table2, tgt, ctx):
    info = plsc.get_sparse_core_info()
    nc, ns = info.num_cores, info.num_subcores
    nw = nc * ns
    b = tgt.shape[0]
    per_w = b // nw
    n_ch = per_w // _CHUNK
    mesh = plsc.VectorSubcoreMesh(core_axis_name="c", subcore_axis_name="s")

    @functools.partial(
        pl.kernel,
        mesh=mesh,
        compiler_params=pltpu.CompilerParams(use_tc_tiling_on_sc=False),
        out_type=jax.ShapeDtypeStruct((b, 4 * _EMB), jnp.float32),
        scratch_types=[
            pltpu.VMEM((per_w,), jnp.int32),
            pltpu.VMEM((per_w,), jnp.int32),
            pltpu.VMEM((_CHUNK, 2 * _EMB), jnp.float32),
            pltpu.VMEM((_CHUNK, 2 * _EMB), jnp.float32),
            pltpu.VMEM((_CHUNK, 2 * _EMB), jnp.float32),
            pltpu.VMEM((_CHUNK, 2 * _EMB), jnp.float32),
            pltpu.SemaphoreType.DMA,
            pltpu.SemaphoreType.DMA,
            pltpu.SemaphoreType.DMA,
            pltpu.SemaphoreType.DMA,
        ],
    )
    def k(tgt_hbm, ctx_hbm, table_hbm, out_hbm, ti_v, ci_v,
          rows_a, rows_b, rows_c, rows_d, s_a, s_b, s_c, s_d):
        wid = lax.axis_index("s") * nc + lax.axis_index("c")
        base = wid * per_w
        pltpu.sync_copy(tgt_hbm.at[pl.ds(base, per_w)], ti_v)
        pltpu.sync_copy(ctx_hbm.at[pl.ds(base, per_w)], ci_v)
        iota16 = lax.iota(jnp.int32, 16)
        _ = iota16

        def halve(g, carry):
            ti_v[pl.ds(g * 16, 16)] = lax.shift_right_logical(
                ti_v[pl.ds(g * 16, 16)], 1)
            ci_v[pl.ds(g * 16, 16)] = lax.shift_right_logical(
                ci_v[pl.ds(g * 16, 16)], 1)
            return carry

        lax.fori_loop(0, per_w // 16, halve, jnp.int32(0))

        bufs = ((rows_a, s_a), (rows_b, s_b), (rows_c, s_c), (rows_d, s_d))
        copies = [None] * 4
        n_items = 2 * n_ch
        for c in range(n_items + 2):
            if c < n_items:
                ch, side = c // 2, c % 2
                idx_ref = (ti_v, ci_v)[side]
                rows, sem = bufs[c % 4]
                copies[c % 4] = pltpu.async_copy(
                    table_hbm.at[idx_ref.at[pl.ds(ch * _CHUNK, _CHUNK)]],
                    rows, sem)
            if c >= 2:
                p = c - 2
                ch, side = p // 2, p % 2
                rows, _unused = bufs[p % 4]
                copies[p % 4].wait()
                pltpu.sync_copy(
                    rows,
                    out_hbm.at[pl.ds(base + ch * _CHUNK, _CHUNK),
                               pl.ds(side * 2 * _EMB, 2 * _EMB)])

    return k(tgt, ctx, table2)


def _tc_mlp(xy, px, py, w1, b1, w2, b2, w3, b3):
    blk = 1024
    n_blk = _BATCH // blk

    def body(xy_ref, px_ref, py_ref, w1_ref, b1_ref, w2_ref, b2_ref, w3_ref,
             b3_ref, o_ref):
        xodd = px_ref[...] == 1
        yodd = py_ref[...] == 1
        x = jnp.where(xodd, xy_ref[:, _EMB:2 * _EMB], xy_ref[:, :_EMB])
        y = jnp.where(yodd, xy_ref[:, 3 * _EMB:], xy_ref[:, 2 * _EMB:3 * _EMB])
        shared = x * y
        h1 = jnp.maximum(
            jnp.dot(shared, w1_ref[...], preferred_element_type=jnp.float32)
            + b1_ref[...], 0.0)
        h2 = jnp.maximum(
            jnp.dot(h1, w2_ref[...], preferred_element_type=jnp.float32)
            + b2_ref[...], 0.0)
        z = jnp.dot(h2, w3_ref[...], preferred_element_type=jnp.float32) \
            + b3_ref[...]
        o_ref[...] = jax.nn.sigmoid(z)

    zero2 = lambda i: (0, 0)
    return pl.pallas_call(
        body,
        grid=(n_blk,),
        in_specs=[
            pl.BlockSpec((blk, 4 * _EMB), lambda i: (i, 0)),
            pl.BlockSpec((blk, 1), lambda i: (i, 0)),
            pl.BlockSpec((blk, 1), lambda i: (i, 0)),
            pl.BlockSpec((_EMB, 64), zero2),
            pl.BlockSpec((1, 64), zero2),
            pl.BlockSpec((64, 32), zero2),
            pl.BlockSpec((1, 32), zero2),
            pl.BlockSpec((32, 1), zero2),
            pl.BlockSpec((1, 1), zero2),
        ],
        out_specs=pl.BlockSpec((blk, 1), lambda i: (i, 0)),
        out_shape=jax.ShapeDtypeStruct((_BATCH, 1), jnp.float32),
    )(xy, px, py, w1, b1, w2, b2, w3, b3)


def kernel(target_word, context_word, table, W1, b1, W2, b2, W3, b3):
    tgt = target_word.astype(jnp.int32)
    ctx = context_word.astype(jnp.int32)
    table2 = jnp.reshape(table, (-1, 2 * _EMB))
    xy = _sc_gather(table2, tgt, ctx)
    out = _tc_mlp(xy, (tgt & 1).reshape(-1, 1), (ctx & 1).reshape(-1, 1),
                  W1, b1.reshape(1, -1), W2, b2.reshape(1, -1), W3,
                  b3.reshape(1, 1))
    return jnp.reshape(out, (-1,))


# speedup vs baseline: 2.3999x; 1.0294x over previous
import functools

import jax
import jax.numpy as jnp
from jax import lax
from jax.experimental import pallas as pl
from jax.experimental.pallas import tpu as pltpu
from jax.experimental.pallas import tpu_sc as plsc

_BATCH = 16384
_EMB = 64
_CHUNK = 128


def _sc_gather(table2, tgt, ctx):
    info = plsc.get_sparse_core_info()
    nc, ns = info.num_cores, info.num_subcores
    nw = nc * ns
    b = tgt.shape[0]
    per_w = b // nw
    n_ch = per_w // _CHUNK
    mesh = plsc.VectorSubcoreMesh(core_axis_name="c", subcore_axis_name="s")

    @functools.partial(
        pl.kernel,
        mesh=mesh,
        compiler_params=pltpu.CompilerParams(),
        out_type=jax.ShapeDtypeStruct((b, 4 * _EMB), jnp.float32),
        scratch_types=[
            pltpu.VMEM((2 * per_w,), jnp.int32),
            pltpu.VMEM((2 * per_w,), jnp.int32),
            pltpu.VMEM((_CHUNK, 2 * _EMB), jnp.float32),
            pltpu.VMEM((_CHUNK, 2 * _EMB), jnp.float32),
            pltpu.VMEM((_CHUNK, 2 * _EMB), jnp.float32),
            pltpu.VMEM((_CHUNK, 2 * _EMB), jnp.float32),
            pltpu.SemaphoreType.DMA,
            pltpu.SemaphoreType.DMA,
            pltpu.SemaphoreType.DMA,
            pltpu.SemaphoreType.DMA,
        ],
    )
    def k(tgt_hbm, ctx_hbm, table_hbm, out_hbm, ti_v, ci_v,
          rows_a, rows_b, rows_c, rows_d, s_a, s_b, s_c, s_d):
        wid = lax.axis_index("s") * nc + lax.axis_index("c")
        base = wid * per_w
        abase = (wid // 2) * (2 * per_w)
        off = (wid % 2) * per_w
        pltpu.sync_copy(tgt_hbm.at[pl.ds(abase, 2 * per_w)], ti_v)
        pltpu.sync_copy(ctx_hbm.at[pl.ds(abase, 2 * per_w)], ci_v)

        def halve(g, carry):
            ti_v[pl.ds(off + g * 16, 16)] = lax.shift_right_logical(
                ti_v[pl.ds(off + g * 16, 16)], 1)
            ci_v[pl.ds(off + g * 16, 16)] = lax.shift_right_logical(
                ci_v[pl.ds(off + g * 16, 16)], 1)
            return carry

        lax.fori_loop(0, per_w // 16, halve, jnp.int32(0))

        bufs = ((rows_a, s_a), (rows_b, s_b), (rows_c, s_c), (rows_d, s_d))
        copies = [None] * 4
        n_items = 2 * n_ch
        for c in range(n_items + 2):
            if c < n_items:
                ch, side = c // 2, c % 2
                idx_ref = (ti_v, ci_v)[side]
                rows, sem = bufs[c % 4]
                copies[c % 4] = pltpu.async_copy(
                    table_hbm.at[idx_ref.at[pl.ds(off + ch * _CHUNK, _CHUNK)]],
                    rows, sem)
            if c >= 2:
                p = c - 2
                ch, side = p // 2, p % 2
                rows, _unused = bufs[p % 4]
                copies[p % 4].wait()
                pltpu.sync_copy(
                    rows,
                    out_hbm.at[pl.ds(base + ch * _CHUNK, _CHUNK),
                               pl.ds(side * 2 * _EMB, 2 * _EMB)])

    return k(tgt, ctx, table2)


def _tc_mlp(xy, px, py, w1, b1, w2, b2, w3, b3):
    blk = 1024
    n_blk = _BATCH // blk

    def body(xy_ref, px_ref, py_ref, w1_ref, b1_ref, w2_ref, b2_ref, w3_ref,
             b3_ref, o_ref):
        xodd = px_ref[...] == 1
        yodd = py_ref[...] == 1
        x = jnp.where(xodd, xy_ref[:, _EMB:2 * _EMB], xy_ref[:, :_EMB])
        y = jnp.where(yodd, xy_ref[:, 3 * _EMB:], xy_ref[:, 2 * _EMB:3 * _EMB])
        shared = x * y
        h1 = jnp.maximum(
            jnp.dot(shared, w1_ref[...], preferred_element_type=jnp.float32)
            + b1_ref[...], 0.0)
        h2 = jnp.maximum(
            jnp.dot(h1, w2_ref[...], preferred_element_type=jnp.float32)
            + b2_ref[...], 0.0)
        z = jnp.dot(h2, w3_ref[...], preferred_element_type=jnp.float32) \
            + b3_ref[...]
        o_ref[...] = jax.nn.sigmoid(z)

    zero2 = lambda i: (0, 0)
    return pl.pallas_call(
        body,
        grid=(n_blk,),
        in_specs=[
            pl.BlockSpec((blk, 4 * _EMB), lambda i: (i, 0)),
            pl.BlockSpec((blk, 1), lambda i: (i, 0)),
            pl.BlockSpec((blk, 1), lambda i: (i, 0)),
            pl.BlockSpec((_EMB, 64), zero2),
            pl.BlockSpec((1, 64), zero2),
            pl.BlockSpec((64, 32), zero2),
            pl.BlockSpec((1, 32), zero2),
            pl.BlockSpec((32, 1), zero2),
            pl.BlockSpec((1, 1), zero2),
        ],
        out_specs=pl.BlockSpec((blk, 1), lambda i: (i, 0)),
        out_shape=jax.ShapeDtypeStruct((_BATCH, 1), jnp.float32),
    )(xy, px, py, w1, b1, w2, b2, w3, b3)


def kernel(target_word, context_word, table, W1, b1, W2, b2, W3, b3):
    tgt = target_word.astype(jnp.int32)
    ctx = context_word.astype(jnp.int32)
    table2 = jnp.reshape(table, (-1, 2 * _EMB))
    xy = _sc_gather(table2, tgt, ctx)
    out = _tc_mlp(xy, (tgt & 1).reshape(-1, 1), (ctx & 1).reshape(-1, 1),
                  W1, b1.reshape(1, -1), W2, b2.reshape(1, -1), W3,
                  b3.reshape(1, 1))
    return jnp.reshape(out, (-1,))


# speedup vs baseline: 2.4796x; 1.0332x over previous
import functools

import jax
import jax.numpy as jnp
from jax import lax
from jax.experimental import pallas as pl
from jax.experimental.pallas import tpu as pltpu
from jax.experimental.pallas import tpu_sc as plsc

_BATCH = 16384
_EMB = 64
_CHUNK = 128


def _sc_gather(table, tgt, ctx):
    info = plsc.get_sparse_core_info()
    nc, ns = info.num_cores, info.num_subcores
    nw = nc * ns
    b = tgt.shape[0]
    per_w = b // nw
    n_ch = per_w // _CHUNK
    mesh = plsc.VectorSubcoreMesh(core_axis_name="c", subcore_axis_name="s")

    @functools.partial(
        pl.kernel,
        mesh=mesh,
        compiler_params=pltpu.CompilerParams(use_tc_tiling_on_sc=False),
        out_type=jax.ShapeDtypeStruct((b, 2 * _EMB), jnp.float32),
        scratch_types=[
            pltpu.VMEM((per_w,), jnp.int32),
            pltpu.VMEM((per_w,), jnp.int32),
            pltpu.VMEM((_CHUNK, _EMB), jnp.float32),
            pltpu.VMEM((_CHUNK, _EMB), jnp.float32),
            pltpu.VMEM((_CHUNK, _EMB), jnp.float32),
            pltpu.VMEM((_CHUNK, _EMB), jnp.float32),
            pltpu.SemaphoreType.DMA,
            pltpu.SemaphoreType.DMA,
            pltpu.SemaphoreType.DMA,
            pltpu.SemaphoreType.DMA,
        ],
    )
    def k(tgt_hbm, ctx_hbm, table_hbm, out_hbm, ti_v, ci_v,
          rows_a, rows_b, rows_c, rows_d, s_a, s_b, s_c, s_d):
        wid = lax.axis_index("s") * nc + lax.axis_index("c")
        base = wid * per_w
        pltpu.sync_copy(tgt_hbm.at[pl.ds(base, per_w)], ti_v)
        pltpu.sync_copy(ctx_hbm.at[pl.ds(base, per_w)], ci_v)
        bufs = ((rows_a, s_a), (rows_b, s_b), (rows_c, s_c), (rows_d, s_d))
        copies = [None] * 4
        n_items = 2 * n_ch
        for c in range(n_items + 2):
            if c < n_items:
                ch, side = c // 2, c % 2
                idx_ref = (ti_v, ci_v)[side]
                rows, sem = bufs[c % 4]
                copies[c % 4] = pltpu.async_copy(
                    table_hbm.at[idx_ref.at[pl.ds(ch * _CHUNK, _CHUNK)]],
                    rows, sem)
            if c >= 2:
                p = c - 2
                ch, side = p // 2, p % 2
                rows, _ = bufs[p % 4]
                copies[p % 4].wait()
                pltpu.sync_copy(
                    rows,
                    out_hbm.at[pl.ds(base + ch * _CHUNK, _CHUNK),
                               pl.ds(side * _EMB, _EMB)])

    return k(tgt, ctx, table)


def _tc_mlp(xy, w1, b1, w2, b2, w3, b3):
    blk = 1024
    n_blk = _BATCH // blk

    def body(xy_ref, w1_ref, b1_ref, w2_ref, b2_ref, w3_ref, b3_ref, o_ref):
        shared = xy_ref[:, :_EMB] * xy_ref[:, _EMB:]
        h1 = jnp.maximum(
            jnp.dot(shared, w1_ref[...], preferred_element_type=jnp.float32)
            + b1_ref[...], 0.0)
        h2 = jnp.maximum(
            jnp.dot(h1, w2_ref[...], preferred_element_type=jnp.float32)
            + b2_ref[...], 0.0)
        z = jnp.dot(h2, w3_ref[...], preferred_element_type=jnp.float32) \
            + b3_ref[...]
        o_ref[...] = jax.nn.sigmoid(z)

    zero2 = lambda i: (0, 0)
    return pl.pallas_call(
        body,
        grid=(n_blk,),
        in_specs=[
            pl.BlockSpec((blk, 2 * _EMB), lambda i: (i, 0)),
            pl.BlockSpec((_EMB, 64), zero2),
            pl.BlockSpec((1, 64), zero2),
            pl.BlockSpec((64, 32), zero2),
            pl.BlockSpec((1, 32), zero2),
            pl.BlockSpec((32, 1), zero2),
            pl.BlockSpec((1, 1), zero2),
        ],
        out_specs=pl.BlockSpec((blk, 1), lambda i: (i, 0)),
        out_shape=jax.ShapeDtypeStruct((_BATCH, 1), jnp.float32),
    )(xy, w1, b1, w2, b2, w3, b3)


def kernel(target_word, context_word, table, W1, b1, W2, b2, W3, b3):
    xy = _sc_gather(table, target_word.astype(jnp.int32),
                    context_word.astype(jnp.int32))
    out = _tc_mlp(xy, W1, b1.reshape(1, -1), W2, b2.reshape(1, -1), W3,
                  b3.reshape(1, 1))
    return jnp.reshape(out, (-1,))
